# baseline (speedup 1.0000x reference)
import jax
import jax.numpy as jnp
from jax.experimental import pallas as pl
from jax.experimental.pallas import tpu as pltpu


def _body(perm_ref, act_ref, fill_ref, x1_ref, x2_ref, y_ref, mask_ref, w_ref,
          b_ref, out_ref, wy_cache):
    i = pl.program_id(0)
    pi = perm_ref[i]
    a = act_ref[i]
    s_al = pl.multiple_of((i // 8) * 8, 8)
    off = i - (i // 8) * 8

    @pl.when(fill_ref[i] == 1)
    def _fill_cache():
        yblk = y_ref[pl.ds(s_al, 8), :]
        wy8 = jax.lax.dot_general(
            yblk, w_ref[0], (((1,), (0,)), ((), ())),
            preferred_element_type=jnp.float32)
        wy_cache[...] = (wy8 + b_ref[a])[:, None, :]

    wy = wy_cache[off]
    s1 = jax.lax.dot_general(
        wy, x1_ref[0, 0], (((1,), (1,)), ((), ())),
        preferred_element_type=jnp.float32)
    s2 = jax.lax.dot_general(
        wy, x2_ref[0, 0], (((1,), (1,)), ((), ())),
        preferred_element_type=jnp.float32)
    out_ref[pi] = jnp.concatenate([s1, s2], axis=1)

    @pl.when(i == pl.num_programs(0) - 1)
    def _epilogue():
        sfull = out_ref[:, 0, :]
        sfull = jnp.where(mask_ref[:, 0, :] != 0, -jnp.inf, sfull)
        m = jnp.max(sfull, axis=-1, keepdims=True)
        sh = sfull - m
        lse = jnp.log(jnp.sum(jnp.exp(sh), axis=-1, keepdims=True))
        out_ref[:, 0, :] = sh - lse


def kernel(x, y, x_mask, actions, weight, bias):
    B, L, X = x.shape
    A, Y, _ = weight.shape
    actions = actions.astype(jnp.int32)
    perm = jnp.argsort(actions).astype(jnp.int32)
    sorted_act = jnp.take(actions, perm)
    idx = jnp.arange(B, dtype=jnp.int32)
    is_break = jnp.concatenate(
        [jnp.ones((1,), bool), sorted_act[1:] != sorted_act[:-1]])
    fill = (is_break | (idx % 8 == 0)).astype(jnp.int32)
    y_sorted = jnp.take(y, perm, axis=0)
    x4 = x.reshape(B, 2, L // 2, X)
    mask_i32 = x_mask.astype(jnp.int32).reshape(B, 1, L)
    bias3 = bias.reshape(A, 1, X)

    grid_spec = pltpu.PrefetchScalarGridSpec(
        num_scalar_prefetch=3,
        grid=(B,),
        in_specs=[
            pl.BlockSpec((1, 1, L // 2, X),
                         lambda i, perm, act, fill: (perm[i], 0, 0, 0)),
            pl.BlockSpec((1, 1, L // 2, X),
                         lambda i, perm, act, fill: (perm[i], 1, 0, 0)),
            pl.BlockSpec((B, Y), lambda i, perm, act, fill: (0, 0)),
            pl.BlockSpec((B, 1, L), lambda i, perm, act, fill: (0, 0, 0)),
            pl.BlockSpec((1, Y, X), lambda i, perm, act, fill: (act[i], 0, 0)),
            pl.BlockSpec((A, 1, X), lambda i, perm, act, fill: (0, 0, 0)),
        ],
        out_specs=pl.BlockSpec((B, 1, L), lambda i, perm, act, fill: (0, 0, 0)),
        scratch_shapes=[pltpu.VMEM((8, 1, X), jnp.float32)],
    )
    out = pl.pallas_call(
        _body,
        grid_spec=grid_spec,
        out_shape=jax.ShapeDtypeStruct((B, 1, L), jnp.float32),
        compiler_params=pltpu.CompilerParams(
            dimension_semantics=("arbitrary",),
        ),
        name="bilinear_seq_attn",
    )(perm, sorted_act, fill, x4, x4, y_sorted, mask_i32, weight, bias3)
    return out.reshape(B, L)


# speedup vs baseline: 1.2088x; 1.2088x over previous
import jax
import jax.numpy as jnp
from jax.experimental import pallas as pl
from jax.experimental.pallas import tpu as pltpu


def _wy_body(act_ref, y_ref, w_ref, binit_ref, wy_ref):
    a = pl.program_id(0)

    @pl.when(a == 0)
    def _init():
        wy_ref[...] = binit_ref[...]

    sel = jnp.where(act_ref[...] == a, y_ref[...], 0.0)
    wy_ref[...] += jax.lax.dot_general(
        sel, w_ref[0], (((1,), (0,)), ((), ())),
        preferred_element_type=jnp.float32)


def _attn_body(x1_ref, x2_ref, wy_ref, mask_ref, out_ref):
    j = pl.program_id(0)
    wyblk = wy_ref[0]
    halves = []
    for xr in (x1_ref, x2_ref):
        rows = []
        for k in range(4):
            rows.append(jax.lax.dot_general(
                wyblk[k:k + 1, :], xr[0, k, 0], (((1,), (1,)), ((), ())),
                preferred_element_type=jnp.float32))
        halves.append(jnp.concatenate(rows, axis=0))
    out_ref[j] = jnp.concatenate(halves, axis=1)

    @pl.when(j == pl.num_programs(0) - 1)
    def _epilogue():
        s = out_ref[...]
        s = jnp.where(mask_ref[...] != 0, -jnp.inf, s)
        m = jnp.max(s, axis=-1, keepdims=True)
        sh = s - m
        lse = jnp.log(jnp.sum(jnp.exp(sh), axis=-1, keepdims=True))
        out_ref[...] = sh - lse


def kernel(x, y, x_mask, actions, weight, bias):
    B, L, X = x.shape
    A, Y, _ = weight.shape
    actions = actions.astype(jnp.int32)
    act2d = actions.reshape(B, 1)
    bias_g = jnp.take(bias, actions, axis=0)

    wy = pl.pallas_call(
        _wy_body,
        grid=(A,),
        in_specs=[
            pl.BlockSpec((B, 1), lambda a: (0, 0)),
            pl.BlockSpec((B, Y), lambda a: (0, 0)),
            pl.BlockSpec((1, Y, X), lambda a: (a, 0, 0)),
            pl.BlockSpec((B, X), lambda a: (0, 0)),
        ],
        out_specs=pl.BlockSpec((B, X), lambda a: (0, 0)),
        out_shape=jax.ShapeDtypeStruct((B, X), jnp.float32),
        compiler_params=pltpu.CompilerParams(
            dimension_semantics=("arbitrary",),
        ),
        name="wy_accumulate",
    )(act2d, y, weight, bias_g)

    G = B // 4
    x5 = x.reshape(G, 4, 2, L // 2, X)
    wy4 = wy.reshape(G, 4, X)
    mask4 = x_mask.astype(jnp.int32).reshape(G, 4, L)

    out = pl.pallas_call(
        _attn_body,
        grid=(G,),
        in_specs=[
            pl.BlockSpec((1, 4, 1, L // 2, X), lambda j: (j, 0, 0, 0, 0)),
            pl.BlockSpec((1, 4, 1, L // 2, X), lambda j: (j, 0, 1, 0, 0)),
            pl.BlockSpec((1, 4, X), lambda j: (j, 0, 0)),
            pl.BlockSpec((G, 4, L), lambda j: (0, 0, 0)),
        ],
        out_specs=pl.BlockSpec((G, 4, L), lambda j: (0, 0, 0)),
        out_shape=jax.ShapeDtypeStruct((G, 4, L), jnp.float32),
        compiler_params=pltpu.CompilerParams(
            dimension_semantics=("arbitrary",),
            vmem_limit_bytes=52 * 1024 * 1024,
        ),
        name="bilinear_scores_softmax",
    )(x5, x5, wy4, mask4)
    return out.reshape(B, L)
